# baseline (speedup 1.0000x reference)
import functools

import jax
import jax.numpy as jnp
from jax import lax
from jax.experimental import pallas as pl
from jax.experimental.pallas import tpu as pltpu
from jax.experimental.pallas import tpu_sc as plsc
import numpy as np

B, V, S, F, K = 4, 1024, 16, 128, 16
MAXD = float(np.finfo(np.float32).max)
R = 256


def _topk_body(act_ref, rows_ref, cols_ref, vals_ref, idx_ref):
    b = pl.program_id(0)
    rblk = pl.program_id(1)
    act = act_ref[0, 0, 0]
    cr = rows_ref[0]
    cc = cols_ref[0]

    def _tree8(h):
        return ((h[0] + h[4]) + (h[2] + h[6])) + ((h[1] + h[5]) + (h[3] + h[7]))

    NT = V // 128
    lane = lax.broadcasted_iota(jnp.int32, (R, 128), 1)
    row = rblk * R + lax.broadcasted_iota(jnp.int32, (R, 128), 0)
    inf = jnp.float32(jnp.inf)
    row_ok = row < act

    dist_t = []
    for t in range(NT):
        a = [
            jnp.abs(cr[:, s : s + 1] - cc[s : s + 1, t * 128 : (t + 1) * 128])
            for s in range(S)
        ]
        d = _tree8(a[0:8]) + _tree8(a[8:16])
        colj = t * 128 + lane
        d = jnp.where(row_ok & (colj < act), d, MAXD)
        d = jnp.where(row == colj, inf, d)
        dist_t.append(d)

    rowv = rblk * R + lax.broadcasted_iota(jnp.int32, (R, 1), 0)
    inact = rowv >= act
    vals_cols = []
    idx_cols = []
    mi = None
    for _p in range(K - 1):
        mvals = []
        for t in range(NT):
            d = dist_t[t]
            if mi is not None:
                d = jnp.where(t * 128 + lane == mi, inf, d)
                dist_t[t] = d
            mvals.append(d)
        mts = list(range(NT))
        while len(mvals) > 1:
            nxt_v, nxt_t = [], []
            for i in range(0, len(mvals), 2):
                va, vb = mvals[i], mvals[i + 1]
                lt = vb < va
                nxt_v.append(jnp.minimum(va, vb))
                nxt_t.append(jnp.where(lt, mts[i + 1], mts[i]))
            mvals, mts = nxt_v, nxt_t
        mkey, mt = mvals[0], mts[0]
        mv = jnp.min(mkey, axis=1, keepdims=True)
        mi = jnp.min(
            jnp.where(mkey == mv, mt * 128 + lane, V), axis=1, keepdims=True
        )
        vals_cols.append(jnp.where(inact, MAXD, mv))
        idx_cols.append(jnp.where(inact, _p + 1, mi))
    vals_ref[0] = jnp.concatenate(
        [jnp.reshape(v, (1, R)) for v in vals_cols], axis=0
    )
    idx_ref[0] = jnp.concatenate(
        [jnp.reshape(i, (1, R)) for i in idx_cols], axis=0
    )


def _topk_call(coords, coords_t, active):
    nb = coords.shape[0]
    return pl.pallas_call(
        _topk_body,
        grid=(nb, V // R),
        in_specs=[
            pl.BlockSpec((1, 1, 1), lambda b, r: (b, 0, 0), memory_space=pltpu.SMEM),
            pl.BlockSpec((1, R, S), lambda b, r: (b, r, 0)),
            pl.BlockSpec((1, S, V), lambda b, r: (b, 0, 0)),
        ],
        out_specs=[
            pl.BlockSpec((1, K - 1, R), lambda b, r: (b, 0, r)),
            pl.BlockSpec((1, K - 1, R), lambda b, r: (b, 0, r)),
        ],
        out_shape=[
            jax.ShapeDtypeStruct((nb, K - 1, V), jnp.float32),
            jax.ShapeDtypeStruct((nb, K - 1, V), jnp.int32),
        ],
        compiler_params=pltpu.CompilerParams(
            dimension_semantics=("parallel", "parallel"),
        ),
    )(active.reshape(nb, 1, 1), coords, coords_t)


NIDX = B * V * (K - 1)
NC, NS = 2, 16
NW = NC * NS
BPW = NIDX // NW
GCH = 120
GNCH = BPW // GCH


def _gather_call(table, idx_flat):
    mesh = plsc.VectorSubcoreMesh(core_axis_name="c", subcore_axis_name="s")

    @functools.partial(
        pl.kernel,
        mesh=mesh,
        out_type=jax.ShapeDtypeStruct((NIDX, F), jnp.float32),
        scratch_types=[
            pltpu.VMEM((BPW,), jnp.int32),
            pltpu.VMEM((GCH, F), jnp.float32),
            pltpu.VMEM((GCH, F), jnp.float32),
            pltpu.SemaphoreType.DMA,
            pltpu.SemaphoreType.DMA,
        ],
    )
    def gk(table_hbm, idx_hbm, out_hbm, idx_v, buf0, buf1, sem0, sem1):
        wid = lax.axis_index("s") * NC + lax.axis_index("c")
        base = wid * BPW
        pltpu.sync_copy(idx_hbm.at[pl.ds(base, BPW)], idx_v)
        bufs = (buf0, buf1)
        sems = (sem0, sem1)
        copies = [None, None]
        for c in range(GNCH + 1):
            if c < GNCH:
                copies[c % 2] = pltpu.async_copy(
                    table_hbm.at[idx_v.at[pl.ds(c * GCH, GCH)]],
                    bufs[c % 2],
                    sems[c % 2],
                )
            if c > 0:
                copies[(c - 1) % 2].wait()
                pltpu.sync_copy(
                    bufs[(c - 1) % 2], out_hbm.at[pl.ds(base + (c - 1) * GCH, GCH)]
                )

    return gk(table, idx_flat)


def kernel(coordinates, features, active_vertices):
    coords_t = jnp.transpose(coordinates, (0, 2, 1))
    ndT, idxT = _topk_call(coordinates, coords_t, active_vertices)
    idx_flat = (idxT + (jnp.arange(B, dtype=jnp.int32) * V)[:, None, None]).reshape(
        NIDX
    )
    nf_lin = _gather_call(features.reshape(B * V, F), idx_flat)
    neighbour_features = jnp.transpose(
        nf_lin.reshape(B, K - 1, V, F), (0, 2, 1, 3)
    )
    neighbour_distances = jnp.transpose(ndT, (0, 2, 1))
    return (neighbour_distances, neighbour_features)


# speedup vs baseline: 1.0262x; 1.0262x over previous
import functools

import jax
import jax.numpy as jnp
from jax import lax
from jax.experimental import pallas as pl
from jax.experimental.pallas import tpu as pltpu
from jax.experimental.pallas import tpu_sc as plsc
import numpy as np

B, V, S, F, K = 4, 1024, 16, 128, 16
MAXD = float(np.finfo(np.float32).max)
R = 256


def _topk_body(act_ref, rows_ref, cols_ref, vals_ref, idx_ref):
    b = pl.program_id(0)
    rblk = pl.program_id(1)
    act = act_ref[0, 0, 0]
    cr = rows_ref[0]
    cc = cols_ref[0]

    def _tree8(h):
        return ((h[0] + h[4]) + (h[2] + h[6])) + ((h[1] + h[5]) + (h[3] + h[7]))

    NT = V // 128
    lane = lax.broadcasted_iota(jnp.int32, (R, 128), 1)
    row = rblk * R + lax.broadcasted_iota(jnp.int32, (R, 128), 0)
    inf = jnp.float32(jnp.inf)
    row_ok = row < act

    dist_t = []
    for t in range(NT):
        a = [
            jnp.abs(cr[:, s : s + 1] - cc[s : s + 1, t * 128 : (t + 1) * 128])
            for s in range(S)
        ]
        d = _tree8(a[0:8]) + _tree8(a[8:16])
        colj = t * 128 + lane
        d = jnp.where(row_ok & (colj < act), d, MAXD)
        d = jnp.where(row == colj, inf, d)
        dist_t.append(d)

    rowv = rblk * R + lax.broadcasted_iota(jnp.int32, (R, 1), 0)
    inact = rowv >= act
    vals_cols = []
    idx_cols = []
    mi = None
    for _p in range(K - 1):
        mvals = []
        for t in range(NT):
            d = dist_t[t]
            if mi is not None:
                d = jnp.where(t * 128 + lane == mi, inf, d)
                dist_t[t] = d
            mvals.append(d)
        mts = list(range(NT))
        while len(mvals) > 1:
            nxt_v, nxt_t = [], []
            for i in range(0, len(mvals), 2):
                va, vb = mvals[i], mvals[i + 1]
                lt = vb < va
                nxt_v.append(jnp.minimum(va, vb))
                nxt_t.append(jnp.where(lt, mts[i + 1], mts[i]))
            mvals, mts = nxt_v, nxt_t
        mkey, mt = mvals[0], mts[0]
        mv = jnp.min(mkey, axis=1, keepdims=True)
        mi = jnp.min(
            jnp.where(mkey == mv, mt * 128 + lane, V), axis=1, keepdims=True
        )
        vals_cols.append(jnp.where(inact, MAXD, mv))
        idx_cols.append(jnp.where(inact, _p + 1, mi))
    vals_ref[0] = jnp.concatenate(
        [jnp.reshape(v, (1, R)) for v in vals_cols], axis=0
    )
    idx_ref[0] = jnp.concatenate(
        [jnp.reshape(i, (1, R)) for i in idx_cols], axis=0
    )


def _topk_call(coords, coords_t, active):
    nb = coords.shape[0]
    return pl.pallas_call(
        _topk_body,
        grid=(nb, V // R),
        in_specs=[
            pl.BlockSpec((1, 1, 1), lambda b, r: (b, 0, 0), memory_space=pltpu.SMEM),
            pl.BlockSpec((1, R, S), lambda b, r: (b, r, 0)),
            pl.BlockSpec((1, S, V), lambda b, r: (b, 0, 0)),
        ],
        out_specs=[
            pl.BlockSpec((1, K - 1, R), lambda b, r: (b, 0, r)),
            pl.BlockSpec((1, K - 1, R), lambda b, r: (b, 0, r)),
        ],
        out_shape=[
            jax.ShapeDtypeStruct((nb, K - 1, V), jnp.float32),
            jax.ShapeDtypeStruct((nb, K - 1, V), jnp.int32),
        ],
        compiler_params=pltpu.CompilerParams(
            dimension_semantics=("parallel", "parallel"),
        ),
    )(active.reshape(nb, 1, 1), coords, coords_t)


NIDX = B * V * (K - 1)
NC, NS = 2, 16
NW = NC * NS
BPW = NIDX // NW
GCH = 384
GNCH = BPW // GCH


def _gather_call(table, idx_flat):
    mesh = plsc.VectorSubcoreMesh(core_axis_name="c", subcore_axis_name="s")

    @functools.partial(
        pl.kernel,
        mesh=mesh,
        out_type=jax.ShapeDtypeStruct((NIDX, F), jnp.float32),
        scratch_types=[
            pltpu.VMEM((BPW,), jnp.int32),
            pltpu.VMEM((GCH, F), jnp.float32),
            pltpu.VMEM((GCH, F), jnp.float32),
            pltpu.SemaphoreType.DMA,
            pltpu.SemaphoreType.DMA,
        ],
    )
    def gk(table_hbm, idx_hbm, out_hbm, idx_v, buf0, buf1, sem0, sem1):
        wid = lax.axis_index("s") * NC + lax.axis_index("c")
        base = wid * BPW
        pltpu.sync_copy(idx_hbm.at[pl.ds(base, BPW)], idx_v)
        bufs = (buf0, buf1)
        sems = (sem0, sem1)
        copies = [None, None]
        for c in range(GNCH + 1):
            if c < GNCH:
                copies[c % 2] = pltpu.async_copy(
                    table_hbm.at[idx_v.at[pl.ds(c * GCH, GCH)]],
                    bufs[c % 2],
                    sems[c % 2],
                )
            if c > 0:
                copies[(c - 1) % 2].wait()
                pltpu.sync_copy(
                    bufs[(c - 1) % 2], out_hbm.at[pl.ds(base + (c - 1) * GCH, GCH)]
                )

    return gk(table, idx_flat)


def kernel(coordinates, features, active_vertices):
    coords_t = jnp.transpose(coordinates, (0, 2, 1))
    ndT, idxT = _topk_call(coordinates, coords_t, active_vertices)
    idx_flat = (idxT + (jnp.arange(B, dtype=jnp.int32) * V)[:, None, None]).reshape(
        NIDX
    )
    nf_lin = _gather_call(features.reshape(B * V, F), idx_flat)
    neighbour_features = jnp.transpose(
        nf_lin.reshape(B, K - 1, V, F), (0, 2, 1, 3)
    )
    neighbour_distances = jnp.transpose(ndT, (0, 2, 1))
    return (neighbour_distances, neighbour_features)


# speedup vs baseline: 1.1742x; 1.1442x over previous
import functools

import jax
import jax.numpy as jnp
from jax import lax
from jax.experimental import pallas as pl
from jax.experimental.pallas import tpu as pltpu
from jax.experimental.pallas import tpu_sc as plsc
import numpy as np

B, V, S, F, K = 4, 1024, 16, 128, 16
MAXD = float(np.finfo(np.float32).max)
R = 512


def _topk_body(act_ref, rows_ref, cols_ref, vals_ref, idx_ref):
    b = pl.program_id(0)
    rblk = pl.program_id(1)
    act = act_ref[0, 0, 0]
    cr = rows_ref[0]
    cc = cols_ref[0]

    def _tree8(h):
        return ((h[0] + h[4]) + (h[2] + h[6])) + ((h[1] + h[5]) + (h[3] + h[7]))

    NT = V // 128
    lane = lax.broadcasted_iota(jnp.int32, (R, 128), 1)
    row = rblk * R + lax.broadcasted_iota(jnp.int32, (R, 128), 0)
    inf = jnp.float32(jnp.inf)
    row_ok = row < act

    dist_t = []
    for t in range(NT):
        a = [
            jnp.abs(cr[:, s : s + 1] - cc[s : s + 1, t * 128 : (t + 1) * 128])
            for s in range(S)
        ]
        d = _tree8(a[0:8]) + _tree8(a[8:16])
        colj = t * 128 + lane
        d = jnp.where(row_ok & (colj < act), d, MAXD)
        d = jnp.where(row == colj, inf, d)
        dist_t.append(d)

    rowv = rblk * R + lax.broadcasted_iota(jnp.int32, (R, 1), 0)
    inact = rowv >= act
    lane_f = lax.broadcasted_iota(jnp.int32, (R, 128), 1).astype(jnp.float32)
    VF = jnp.float32(V)
    vals_cols = []
    idx_cols = []
    mi = None
    for _p in range(K - 1):
        mvals = []
        for t in range(NT):
            d = dist_t[t]
            if mi is not None:
                d = jnp.where(t * 128.0 + lane_f == mi, inf, d)
                dist_t[t] = d
            mvals.append(d)
        mts = [jnp.float32(t) for t in range(NT)]
        while len(mvals) > 1:
            nxt_v, nxt_t = [], []
            for i in range(0, len(mvals), 2):
                va, vb = mvals[i], mvals[i + 1]
                lt = vb < va
                nxt_v.append(jnp.minimum(va, vb))
                nxt_t.append(jnp.where(lt, mts[i + 1], mts[i]))
            mvals, mts = nxt_v, nxt_t
        mkey, mt = mvals[0], mts[0]
        mv = jnp.min(mkey, axis=1, keepdims=True)
        mi = jnp.min(
            jnp.where(mkey == mv, mt * 128.0 + lane_f, VF), axis=1, keepdims=True
        )
        vals_cols.append(jnp.where(inact, MAXD, mv))
        idx_cols.append(jnp.where(inact, _p + 1, mi.astype(jnp.int32)))
    vals_ref[0] = jnp.concatenate(
        [jnp.reshape(v, (1, R)) for v in vals_cols], axis=0
    )
    idx_ref[0] = jnp.concatenate(
        [jnp.reshape(i, (1, R)) for i in idx_cols], axis=0
    )


def _topk_call(coords, coords_t, active):
    nb = coords.shape[0]
    return pl.pallas_call(
        _topk_body,
        grid=(nb, V // R),
        in_specs=[
            pl.BlockSpec((1, 1, 1), lambda b, r: (b, 0, 0), memory_space=pltpu.SMEM),
            pl.BlockSpec((1, R, S), lambda b, r: (b, r, 0)),
            pl.BlockSpec((1, S, V), lambda b, r: (b, 0, 0)),
        ],
        out_specs=[
            pl.BlockSpec((1, K - 1, R), lambda b, r: (b, 0, r)),
            pl.BlockSpec((1, K - 1, R), lambda b, r: (b, 0, r)),
        ],
        out_shape=[
            jax.ShapeDtypeStruct((nb, K - 1, V), jnp.float32),
            jax.ShapeDtypeStruct((nb, K - 1, V), jnp.int32),
        ],
        compiler_params=pltpu.CompilerParams(
            dimension_semantics=("parallel", "parallel"),
        ),
    )(active.reshape(nb, 1, 1), coords, coords_t)


NIDX = B * V * (K - 1)
NC, NS = 2, 16
NW = NC * NS
BPW = NIDX // NW
GCH = 480
GNCH = BPW // GCH


def _gather_call(table, idx_flat):
    mesh = plsc.VectorSubcoreMesh(core_axis_name="c", subcore_axis_name="s")

    @functools.partial(
        pl.kernel,
        mesh=mesh,
        out_type=jax.ShapeDtypeStruct((NIDX, F), jnp.float32),
        scratch_types=[
            pltpu.VMEM((BPW,), jnp.int32),
            pltpu.VMEM((GCH, F), jnp.float32),
            pltpu.VMEM((GCH, F), jnp.float32),
            pltpu.SemaphoreType.DMA,
            pltpu.SemaphoreType.DMA,
        ],
    )
    def gk(table_hbm, idx_hbm, out_hbm, idx_v, buf0, buf1, sem0, sem1):
        wid = lax.axis_index("s") * NC + lax.axis_index("c")
        base = wid * BPW
        pltpu.sync_copy(idx_hbm.at[pl.ds(base, BPW)], idx_v)
        bufs = (buf0, buf1)
        sems = (sem0, sem1)
        copies = [None, None]
        for c in range(GNCH + 1):
            if c < GNCH:
                copies[c % 2] = pltpu.async_copy(
                    table_hbm.at[idx_v.at[pl.ds(c * GCH, GCH)]],
                    bufs[c % 2],
                    sems[c % 2],
                )
            if c > 0:
                copies[(c - 1) % 2].wait()
                pltpu.sync_copy(
                    bufs[(c - 1) % 2], out_hbm.at[pl.ds(base + (c - 1) * GCH, GCH)]
                )

    return gk(table, idx_flat)


def kernel(coordinates, features, active_vertices):
    coords_t = jnp.transpose(coordinates, (0, 2, 1))
    ndT, idxT = _topk_call(coordinates, coords_t, active_vertices)
    idx_flat = (idxT + (jnp.arange(B, dtype=jnp.int32) * V)[:, None, None]).reshape(
        NIDX
    )
    nf_lin = _gather_call(features.reshape(B * V, F), idx_flat)
    neighbour_features = jnp.transpose(
        nf_lin.reshape(B, K - 1, V, F), (0, 2, 1, 3)
    )
    neighbour_distances = jnp.transpose(ndT, (0, 2, 1))
    return (neighbour_distances, neighbour_features)
